# baseline (speedup 1.0000x reference)
import functools

import jax
import jax.numpy as jnp
from jax import lax
from jax.experimental import pallas as pl
from jax.experimental.pallas import tpu as pltpu
from jax.experimental.pallas import tpu_sc as plsc

N_NODE = 1_000_000
DIM = 64
BATCH = 16384
L = 16
NC, NS = 2, 16
NW = NC * NS
BPW = BATCH // NW
CH = 128
NCH = BPW // CH

_mesh = plsc.VectorSubcoreMesh(core_axis_name="c", subcore_axis_name="s")


@functools.partial(
    pl.kernel,
    mesh=_mesh,
    compiler_params=pltpu.CompilerParams(use_tc_tiling_on_sc=False),
    out_type=[
        jax.ShapeDtypeStruct((BATCH, DIM), jnp.float32),
        jax.ShapeDtypeStruct((BATCH, DIM), jnp.float32),
        jax.ShapeDtypeStruct((BATCH,), jnp.float32),
    ],
    scratch_types=[
        pltpu.VMEM((NCH, CH), jnp.int32),
        pltpu.VMEM((NCH, CH), jnp.int32),
        pltpu.VMEM((BPW, DIM), jnp.float32),
        pltpu.VMEM((BPW, DIM), jnp.float32),
        pltpu.VMEM((BPW,), jnp.float32),
        pltpu.VMEM((BPW,), jnp.float32),
        pltpu.SemaphoreType.DMA,
    ],
)
def _gan_kernel(emb, bias, ida, idb, out_a, out_b, out_p,
                idx_a, idx_b, rows_a, rows_b, bias_v, prob_v, sem):
    wid = lax.axis_index("s") * NC + lax.axis_index("c")
    base = wid * BPW

    for j in range(NCH):
        pltpu.sync_copy(ida.at[pl.ds(base + j * CH, CH)], idx_a.at[j])
        pltpu.sync_copy(idb.at[pl.ds(base + j * CH, CH)], idx_b.at[j])

    copies = []
    for j in range(NCH):
        copies.append(pltpu.async_copy(
            emb.at[idx_a.at[j]], rows_a.at[pl.ds(j * CH, CH)], sem))
        copies.append(pltpu.async_copy(
            emb.at[idx_b.at[j]], rows_b.at[pl.ds(j * CH, CH)], sem))
        copies.append(pltpu.async_copy(
            bias.at[idx_b.at[j]], bias_v.at[pl.ds(j * CH, CH)], sem))
    for c in copies:
        c.wait()

    pltpu.sync_copy(rows_a, out_a.at[pl.ds(base, BPW)])
    pltpu.sync_copy(rows_b, out_b.at[pl.ds(base, BPW)])

    lane = lax.iota(jnp.int32, L)
    perms = [(lane ^ sh).reshape(L, 1) for sh in (8, 4, 2, 1)]
    dnums = lax.GatherDimensionNumbers(
        offset_dims=(), collapsed_slice_dims=(0,), start_index_map=(0,))

    def lane_shuffle(v, p):
        return lax.gather(v, p, dnums, (1,),
                          mode=lax.GatherScatterMode.PROMISE_IN_BOUNDS)

    def hsum(v):
        for p in perms:
            v = v + lane_shuffle(v, p)
        return v

    def group_body(g, carry):
        s = jnp.zeros((L,), jnp.float32)
        for k in range(L):
            r = g * L + k
            acc = rows_a[r, pl.ds(0, L)] * rows_b[r, pl.ds(0, L)]
            for c in range(1, DIM // L):
                acc = acc + rows_a[r, pl.ds(c * L, L)] * rows_b[r, pl.ds(c * L, L)]
            s = jnp.where(lane == k, hsum(acc), s)
        s = s + bias_v[pl.ds(g * L, L)]
        p = 1.0 / (1.0 + jnp.exp(-s))
        p = jnp.minimum(jnp.maximum(p, 1e-5), 1.0)
        prob_v[pl.ds(g * L, L)] = p
        return carry

    lax.fori_loop(0, BPW // L, group_body, 0)
    pltpu.sync_copy(prob_v, out_p.at[pl.ds(base, BPW)])


def kernel(embedding_matrix, bias_vector, node_id, node_neighbor_id):
    node_emb, neigh_emb, prob = _gan_kernel(
        embedding_matrix, bias_vector, node_id, node_neighbor_id)
    return (node_emb, neigh_emb, prob)
